# Initial kernel scaffold; baseline (speedup 1.0000x reference)
import functools

import jax
import jax.numpy as jnp
from jax import lax
from jax.experimental import pallas as pl
from jax.experimental.pallas import tpu as pltpu
from jax.experimental.pallas import tpu_sc as plsc

EMBED = 64
B_TOTAL = 4096 * 200
NUM_CORES = 2
NUM_SUBCORES = 16
NUM_WORKERS = NUM_CORES * NUM_SUBCORES
B_PER_W = B_TOTAL // NUM_WORKERS
CHUNK = 128
NBUF = 2
STEPS = B_PER_W // CHUNK

_mesh = plsc.VectorSubcoreMesh(core_axis_name="c", subcore_axis_name="s")


@functools.partial(
    pl.kernel,
    mesh=_mesh,
    out_type=jax.ShapeDtypeStruct((B_TOTAL, EMBED), jnp.float32),
    scratch_types=[
        pltpu.VMEM((B_PER_W,), jnp.int32),
        pltpu.VMEM((NBUF, CHUNK, EMBED), jnp.float32),
        pltpu.SemaphoreType.DMA,
    ],
)
def _embed_lookup(idx_hbm, table_hbm, out_hbm, idx_v, rows_v, gsem):
    wid = lax.axis_index("s") * NUM_CORES + lax.axis_index("c")
    base = wid * B_PER_W

    pltpu.sync_copy(idx_hbm.at[pl.ds(base, B_PER_W)], idx_v)

    def start_gather(step, buf):
        pltpu.make_async_copy(
            table_hbm.at[idx_v.at[pl.ds(step * CHUNK, CHUNK)]],
            rows_v.at[buf],
            gsem,
        ).start()

    def wait_gather(buf):
        pltpu.make_async_copy(
            table_hbm.at[idx_v.at[pl.ds(0, CHUNK)]],
            rows_v.at[buf],
            gsem,
        ).wait()

    for b in range(NBUF):
        start_gather(b, b)

    def body(i, carry):
        t0 = i * NBUF
        for b in range(NBUF):
            t = t0 + b
            wait_gather(b)
            pltpu.sync_copy(
                rows_v.at[b],
                out_hbm.at[pl.ds(base + t * CHUNK, CHUNK)],
            )

            @pl.when(t + NBUF < STEPS)
            def _():
                start_gather(t + NBUF, b)

        return carry

    lax.fori_loop(0, STEPS // NBUF, body, 0)


def kernel(indices, W):
    idx = indices.reshape(-1).astype(jnp.int32)
    out = _embed_lookup(idx, W)
    return out.reshape(indices.shape + (W.shape[1],))


# speedup vs baseline: 3.5769x; 3.5769x over previous
import functools

import jax
import jax.numpy as jnp
from jax import lax
from jax.experimental import pallas as pl
from jax.experimental.pallas import tpu as pltpu
from jax.experimental.pallas import tpu_sc as plsc

EMBED = 64
B_TOTAL = 4096 * 200
NUM_CORES = 2
NUM_SUBCORES = 16
NUM_WORKERS = NUM_CORES * NUM_SUBCORES
B_PER_W = B_TOTAL // NUM_WORKERS
CHUNK = 128
NBUF = 2
STEPS = B_PER_W // CHUNK

_mesh = plsc.VectorSubcoreMesh(core_axis_name="c", subcore_axis_name="s")


@functools.partial(
    pl.kernel,
    mesh=_mesh,
    out_type=jax.ShapeDtypeStruct((B_TOTAL, EMBED), jnp.float32),
    scratch_types=[
        pltpu.VMEM((B_PER_W,), jnp.int32),
        pltpu.VMEM((NBUF, CHUNK, EMBED), jnp.float32),
        pltpu.SemaphoreType.DMA,
    ],
    compiler_params=pltpu.CompilerParams(use_tc_tiling_on_sc=False),
)
def _embed_lookup(idx_hbm, table_hbm, out_hbm, idx_v, rows_v, gsem):
    wid = lax.axis_index("s") * NUM_CORES + lax.axis_index("c")
    base = wid * B_PER_W

    pltpu.sync_copy(idx_hbm.at[pl.ds(base, B_PER_W)], idx_v)

    def start_gather(step, buf):
        pltpu.make_async_copy(
            table_hbm.at[idx_v.at[pl.ds(step * CHUNK, CHUNK)]],
            rows_v.at[buf],
            gsem,
        ).start()

    def wait_gather(buf):
        pltpu.make_async_copy(
            table_hbm.at[idx_v.at[pl.ds(0, CHUNK)]],
            rows_v.at[buf],
            gsem,
        ).wait()

    for b in range(NBUF):
        start_gather(b, b)

    def body(i, carry):
        t0 = i * NBUF
        for b in range(NBUF):
            t = t0 + b
            wait_gather(b)
            pltpu.sync_copy(
                rows_v.at[b],
                out_hbm.at[pl.ds(base + t * CHUNK, CHUNK)],
            )

            @pl.when(t + NBUF < STEPS)
            def _():
                start_gather(t + NBUF, b)

        return carry

    lax.fori_loop(0, STEPS // NBUF, body, 0)


def kernel(indices, W):
    idx = indices.reshape(-1).astype(jnp.int32)
    out = _embed_lookup(idx, W)
    return out.reshape(indices.shape + (W.shape[1],))
